# baseline (speedup 1.0000x reference)
import functools

import jax
import jax.numpy as jnp
from jax import lax
from jax.experimental import pallas as pl
from jax.experimental.pallas import tpu as pltpu

B = 2
T = 2048
D = 1024
E = 8
K = 2
N = B * T

TBLK = 512
NT = N // TBLK


def _fused_dense_kernel(x_ref, logits_ref, w_ref, eb_ref, out_ref, m_s):
    e = pl.program_id(1)

    @pl.when(e == 0)
    def _gate():
        logits = logits_ref[...]
        mx = jnp.max(logits, axis=-1, keepdims=True)
        p = jnp.exp(logits - mx)
        probs = p / jnp.sum(p, axis=-1, keepdims=True)
        iota = lax.broadcasted_iota(jnp.int32, (TBLK, E), 1)
        m1 = jnp.max(probs, axis=-1, keepdims=True)
        i1 = jnp.min(jnp.where(probs == m1, iota, E), axis=-1, keepdims=True)
        p2 = jnp.where(iota == i1, -1.0, probs)
        m2 = jnp.max(p2, axis=-1, keepdims=True)
        i2 = jnp.min(jnp.where(p2 == m2, iota, E), axis=-1, keepdims=True)
        m_s[...] = jnp.where((iota == i1) | (iota == i2), probs, 0.0)
        out_ref[...] = jnp.zeros_like(out_ref)

    iota = lax.broadcasted_iota(jnp.int32, (TBLK, E), 1)
    wcol = jnp.sum(m_s[...] * (iota == e).astype(jnp.float32),
                   axis=-1, keepdims=True)
    y = jnp.dot(x_ref[...], w_ref[0],
                preferred_element_type=jnp.float32,
                precision=lax.Precision.HIGHEST)
    out_ref[...] += (y + eb_ref[0]) * wcol


@jax.jit
def kernel(hidden_states, gate_w, gate_b, expert_w, expert_b):
    x2d = hidden_states.reshape(N, D)
    eb3d = expert_b.reshape(E, 1, D)
    logits = (hidden_states @ gate_w + gate_b).reshape(N, E)

    out = pl.pallas_call(
        _fused_dense_kernel,
        grid=(NT, E),
        in_specs=[
            pl.BlockSpec((TBLK, D), lambda t, e: (t, 0)),
            pl.BlockSpec((TBLK, E), lambda t, e: (t, 0)),
            pl.BlockSpec((1, D, D), lambda t, e: (e, 0, 0)),
            pl.BlockSpec((1, 1, D), lambda t, e: (e, 0, 0)),
        ],
        out_specs=pl.BlockSpec((TBLK, D), lambda t, e: (t, 0)),
        out_shape=jax.ShapeDtypeStruct((N, D), jnp.float32),
        scratch_shapes=[pltpu.VMEM((TBLK, E), jnp.float32)],
    )(x2d, logits, expert_w, eb3d)

    return out.reshape(B, T, D)


# speedup vs baseline: 3.1789x; 3.1789x over previous
import functools

import jax
import jax.numpy as jnp
from jax import lax
from jax.experimental import pallas as pl
from jax.experimental.pallas import tpu as pltpu

B = 2
T = 2048
D = 1024
E = 8
K = 2
N = B * T

TBLK = 512
NT = N // TBLK


def _fused_dense_kernel(x_ref, logits_ref, w_ref, eb_ref, out_ref, m_s):
    e = pl.program_id(1)

    @pl.when(e == 0)
    def _gate():
        logits = logits_ref[...]
        mx = jnp.max(logits, axis=-1, keepdims=True)
        p = jnp.exp(logits - mx)
        probs = p / jnp.sum(p, axis=-1, keepdims=True)
        iota = lax.broadcasted_iota(jnp.int32, (TBLK, E), 1)
        m1 = jnp.max(probs, axis=-1, keepdims=True)
        i1 = jnp.min(jnp.where(probs == m1, iota, E), axis=-1, keepdims=True)
        p2 = jnp.where(iota == i1, -1.0, probs)
        m2 = jnp.max(p2, axis=-1, keepdims=True)
        i2 = jnp.min(jnp.where(p2 == m2, iota, E), axis=-1, keepdims=True)
        m_s[...] = jnp.where((iota == i1) | (iota == i2), probs, 0.0)
        out_ref[...] = jnp.zeros_like(out_ref)

    iota = lax.broadcasted_iota(jnp.int32, (TBLK, E), 1)
    wcol = jnp.sum(m_s[...] * (iota == e).astype(jnp.float32),
                   axis=-1, keepdims=True)
    y = jnp.dot(x_ref[...], w_ref[0],
                preferred_element_type=jnp.float32)
    out_ref[...] += (y + eb_ref[0]) * wcol


@jax.jit
def kernel(hidden_states, gate_w, gate_b, expert_w, expert_b):
    x2d = hidden_states.reshape(N, D)
    eb3d = expert_b.reshape(E, 1, D)
    logits = (hidden_states @ gate_w + gate_b).reshape(N, E)

    out = pl.pallas_call(
        _fused_dense_kernel,
        grid=(NT, E),
        in_specs=[
            pl.BlockSpec((TBLK, D), lambda t, e: (t, 0)),
            pl.BlockSpec((TBLK, E), lambda t, e: (t, 0)),
            pl.BlockSpec((1, D, D), lambda t, e: (e, 0, 0)),
            pl.BlockSpec((1, 1, D), lambda t, e: (e, 0, 0)),
        ],
        out_specs=pl.BlockSpec((TBLK, D), lambda t, e: (t, 0)),
        out_shape=jax.ShapeDtypeStruct((N, D), jnp.float32),
        scratch_shapes=[pltpu.VMEM((TBLK, E), jnp.float32)],
    )(x2d, logits, expert_w, eb3d)

    return out.reshape(B, T, D)
